# baseline (speedup 1.0000x reference)
import functools

import jax
import jax.numpy as jnp
from jax import lax
from jax.experimental import pallas as pl
from jax.experimental.pallas import tpu as pltpu
from jax.experimental.pallas import tpu_sc as plsc

NC = 2
NS = 16
NW = NC * NS
L = 16
BLK = 128
GSTR = 128


def _build_sc_kernel(N, D, E, K, OUT):
    SEGW = -(-OUT // NW)
    SEGW = ((SEGW + 7) // 8) * 8
    OUT_PAD = NW * SEGW
    EP = E + BLK
    DL = D // L
    NM = 2 + K
    NSENT = N
    NPAD = -(-(N + 8) // NS // 8) * 8 * NS
    mesh = plsc.VectorSubcoreMesh(core_axis_name="c", subcore_axis_name="s")

    @functools.partial(
        pl.kernel,
        out_type=jax.ShapeDtypeStruct((OUT_PAD, D), jnp.float32),
        mesh=mesh,
        scratch_types=[
            pltpu.VMEM((48,), jnp.int32),
            pltpu.VMEM((2, NM, BLK), jnp.int32),
            pltpu.VMEM((2, BLK), jnp.int32),
            pltpu.VMEM((2, BLK, D), jnp.float32),
            pltpu.VMEM((SEGW + 1, D), jnp.float32),
            pltpu.VMEM_SHARED((NPAD, D), jnp.float32),
            pltpu.SemaphoreType.DMA,
            pltpu.SemaphoreType.DMA,
        ],
    )
    def _k(feat_h, meta_h, bnd_h, out_h,
           bnd_v, m_v, idx_v, g_v, acc_v, f_sh, sem, sem2):
        cid = lax.axis_index("c")
        sid = lax.axis_index("s")
        wid = sid * NC + cid
        s0 = wid * SEGW

        CH = NPAD // NS
        pltpu.sync_copy(feat_h.at[pl.ds(sid * CH, CH)],
                        f_sh.at[pl.ds(sid * CH, CH)])
        plsc.subcore_barrier()
        pltpu.sync_copy(bnd_h, bnd_v)
        bv = bnd_v[pl.ds(wid, L)]
        e_lo = bv[0]
        e_hi = bv[1]
        e_lo = e_lo - lax.rem(e_lo, BLK)
        nblk = lax.div(e_hi - e_lo + (BLK - 1), BLK)

        neg_inf = jnp.full((L,), -jnp.inf, jnp.float32)

        def init_body(r, _):
            for k in range(DL):
                acc_v[r, pl.ds(k * L, L)] = neg_inf
            return 0

        lax.fori_loop(0, SEGW + 1, init_body, 0)

        def meta_copy(b):
            mslot = lax.rem(b, 2)
            base = pl.multiple_of(e_lo + b * BLK, BLK)
            return pltpu.make_async_copy(
                meta_h.at[:, pl.ds(base, BLK)], m_v.at[mslot], sem2)

        def gather_fire(b):
            mslot = lax.rem(b, 2)
            slot = lax.rem(b, 2)
            meta_copy(b).wait()

            def mask_body(g, _):
                sl = pl.ds(g * L, L)
                acc = m_v[mslot, 2, sl]
                for j in range(3, NM):
                    acc = acc | m_v[mslot, j, sl]
                idx_v[slot, sl] = jnp.where(acc >= 0, m_v[mslot, 1, sl],
                                            NSENT)
                return 0

            lax.fori_loop(0, BLK // L, mask_body, 0)
            for q in range(BLK // GSTR):
                pltpu.make_async_copy(
                    f_sh.at[idx_v.at[slot].at[pl.ds(q * GSTR, GSTR)]],
                    g_v.at[slot].at[pl.ds(q * GSTR, GSTR)],
                    sem).start()

        @pl.when(nblk > 0)
        def _():
            meta_copy(0).start()
            gather_fire(0)

        def block_body(b, _):
            mslot = lax.rem(b, 2)
            slot = lax.rem(b, 2)

            @pl.when(b + 1 < nblk)
            def _():
                meta_copy(b + 1).start()

            for q in range(BLK // GSTR):
                pltpu.make_async_copy(
                    f_sh.at[idx_v.at[slot].at[pl.ds(q * GSTR, GSTR)]],
                    g_v.at[slot].at[pl.ds(q * GSTR, GSTR)],
                    sem).wait()

            @pl.when(b + 1 < nblk)
            def _():
                gather_fire(b + 1)

            def acc_body(g, _):
                sl = pl.ds(g * L, L)
                r0 = m_v[mslot, 0, sl] - s0
                rowv = jnp.where(r0 < 0, SEGW, jnp.minimum(r0, SEGW))
                row0 = rowv[0]
                same = (row0 == rowv[L - 1]) & (row0 < SEGW)

                def fast(_):
                    e0 = g * L
                    for k in range(DL):
                        fsl = pl.ds(k * L, L)
                        m = g_v[slot, e0, fsl]
                        for lane in range(1, L):
                            m = jnp.maximum(m, g_v[slot, e0 + lane, fsl])
                        acc_v[row0, fsl] = jnp.maximum(acc_v[row0, fsl], m)
                    return 0

                def slow(_):
                    for lane in range(L):
                        row = rowv[lane]
                        e = g * L + lane
                        for k in range(DL):
                            fsl = pl.ds(k * L, L)
                            acc_v[row, fsl] = jnp.maximum(
                                acc_v[row, fsl], g_v[slot, e, fsl])
                    return 0

                lax.cond(same, fast, slow, 0)
                return 0

            lax.fori_loop(0, BLK // L, acc_body, 0)
            return 0

        lax.fori_loop(0, nblk, block_body, 0)

        def fin_body(r, _):
            for k in range(DL):
                sl = pl.ds(k * L, L)
                v = acc_v[r, sl]
                acc_v[r, sl] = jnp.where(v == -jnp.inf, 0.0, v)
            return 0

        lax.fori_loop(0, SEGW, fin_body, 0)
        pltpu.sync_copy(acc_v.at[pl.ds(0, SEGW)],
                        out_h.at[pl.ds(pl.multiple_of(s0, 8), SEGW)])

    return _k, SEGW, OUT_PAD, NPAD


def kernel(feat, src_ids, tgt_ids, ntypes, out_size):
    N, D = feat.shape
    E = src_ids.shape[0]
    K = ntypes.shape[1]
    try:
        OUT = int(out_size)
    except (jax.errors.ConcretizationTypeError, TypeError):
        OUT = 2500

    k, SEGW, OUT_PAD, NPAD = _build_sc_kernel(N, D, E, K, OUT)

    feat_ext = jnp.concatenate(
        [feat, jnp.full((NPAD - N, D), -jnp.inf, feat.dtype)], axis=0)
    srcp = jnp.concatenate(
        [src_ids, jnp.full((BLK,), OUT_PAD + BLK, jnp.int32)])
    tgtp = jnp.concatenate([tgt_ids, jnp.zeros((BLK,), jnp.int32)])
    ntp = jnp.concatenate(
        [ntypes.T, jnp.full((K, BLK), -1, jnp.int32)], axis=1)
    meta = jnp.concatenate([srcp[None, :], tgtp[None, :], ntp], axis=0)
    fences = jnp.arange(0, OUT_PAD + SEGW, SEGW, dtype=jnp.int32)[:NW + 1]
    bounds = jnp.searchsorted(src_ids, fences).astype(jnp.int32)
    bounds = jnp.concatenate(
        [bounds, jnp.full((48 - NW - 1,), E, jnp.int32)])

    out = k(feat_ext, meta, bounds)
    return out[:OUT]


# speedup vs baseline: 1.1887x; 1.1887x over previous
import functools

import jax
import jax.numpy as jnp
from jax import lax
from jax.experimental import pallas as pl
from jax.experimental.pallas import tpu as pltpu
from jax.experimental.pallas import tpu_sc as plsc

NC = 2
NS = 16
NW = NC * NS
L = 16
BLK = 128
GSTR = 128


def _build_sc_kernel(N, D, E, K, OUT):
    SEGW = -(-OUT // NW)
    SEGW = ((SEGW + 7) // 8) * 8
    OUT_PAD = NW * SEGW
    EP = E + BLK
    DL = D // L
    NM = 2 + K
    NSENT = N
    NPAD = -(-(N + 8) // 8) * 8
    CHB = NPAD // NS // 8 * 8
    NEX = (NPAD - CHB * NS) // 8
    mesh = plsc.VectorSubcoreMesh(core_axis_name="c", subcore_axis_name="s")

    @functools.partial(
        pl.kernel,
        out_type=jax.ShapeDtypeStruct((OUT_PAD, D), jnp.float32),
        mesh=mesh,
        scratch_types=[
            pltpu.VMEM((48,), jnp.int32),
            pltpu.VMEM((3, NM, BLK), jnp.int32),
            pltpu.VMEM((2, BLK), jnp.int32),
            pltpu.VMEM((2, BLK, D), jnp.float32),
            pltpu.VMEM((SEGW + 1, D), jnp.float32),
            pltpu.VMEM_SHARED((NPAD, D), jnp.float32),
            pltpu.SemaphoreType.DMA,
            pltpu.SemaphoreType.DMA,
        ],
    )
    def _k(feat_h, meta_h, bnd_h, out_h,
           bnd_v, m_v, idx_v, g_v, acc_v, f_sh, sem, sem2):
        cid = lax.axis_index("c")
        sid = lax.axis_index("s")
        wid = sid * NC + cid
        s0 = wid * SEGW

        off = pl.multiple_of(
            CHB * sid + 8 * jnp.minimum(sid, NEX), 8)

        @pl.when(sid < NEX)
        def _():
            pltpu.sync_copy(feat_h.at[pl.ds(off, CHB + 8)],
                            f_sh.at[pl.ds(off, CHB + 8)])

        @pl.when(sid >= NEX)
        def _():
            pltpu.sync_copy(feat_h.at[pl.ds(off, CHB)],
                            f_sh.at[pl.ds(off, CHB)])

        plsc.subcore_barrier()
        pltpu.sync_copy(bnd_h, bnd_v)
        bv = bnd_v[pl.ds(wid, L)]
        e_lo = bv[0]
        e_hi = bv[1]
        e_lo = e_lo - lax.rem(e_lo, BLK)
        nblk = lax.div(e_hi - e_lo + (BLK - 1), BLK)

        neg_inf = jnp.full((L,), -jnp.inf, jnp.float32)

        def init_body(r, _):
            for k in range(DL):
                acc_v[r, pl.ds(k * L, L)] = neg_inf
            return 0

        lax.fori_loop(0, SEGW + 1, init_body, 0)

        def meta_copy(b):
            mslot = lax.rem(b, 3)
            base = pl.multiple_of(e_lo + b * BLK, BLK)
            return pltpu.make_async_copy(
                meta_h.at[:, pl.ds(base, BLK)], m_v.at[mslot], sem2)

        def gather_fire(b, nblk):
            mslot = lax.rem(b, 3)
            slot = lax.rem(b, 2)
            meta_copy(b).wait()

            def mask_body(g, _):
                sl = pl.ds(g * L, L)
                acc = m_v[mslot, 2, sl]
                for j in range(3, NM):
                    acc = acc | m_v[mslot, j, sl]
                idx_v[slot, sl] = jnp.where(acc >= 0, m_v[mslot, 1, sl],
                                            NSENT)
                return 0

            lax.fori_loop(0, BLK // L, mask_body, 0)
            for q in range(BLK // GSTR):
                pltpu.make_async_copy(
                    f_sh.at[idx_v.at[slot].at[pl.ds(q * GSTR, GSTR)]],
                    g_v.at[slot].at[pl.ds(q * GSTR, GSTR)],
                    sem).start()

            @pl.when(b + 1 < nblk)
            def _():
                meta_copy(b + 1).start()

        @pl.when(nblk > 0)
        def _():
            meta_copy(0).start()
            gather_fire(0, nblk)

        def block_body(b, _):
            mslot = lax.rem(b, 3)
            slot = lax.rem(b, 2)

            for q in range(BLK // GSTR):
                pltpu.make_async_copy(
                    f_sh.at[idx_v.at[slot].at[pl.ds(q * GSTR, GSTR)]],
                    g_v.at[slot].at[pl.ds(q * GSTR, GSTR)],
                    sem).wait()

            @pl.when(b + 1 < nblk)
            def _():
                gather_fire(b + 1, nblk)

            def acc_body(g, _):
                sl = pl.ds(g * L, L)
                r0 = m_v[mslot, 0, sl] - s0
                rowv = jnp.where(r0 < 0, SEGW, jnp.minimum(r0, SEGW))
                row0 = rowv[0]
                same = (row0 == rowv[L - 1]) & (row0 < SEGW)

                def fast(_):
                    e0 = g * L
                    for k in range(DL):
                        fsl = pl.ds(k * L, L)
                        m = g_v[slot, e0, fsl]
                        for lane in range(1, L):
                            m = jnp.maximum(m, g_v[slot, e0 + lane, fsl])
                        acc_v[row0, fsl] = jnp.maximum(acc_v[row0, fsl], m)
                    return 0

                def slow(_):
                    for lane in range(L):
                        row = rowv[lane]
                        e = g * L + lane
                        for k in range(DL):
                            fsl = pl.ds(k * L, L)
                            acc_v[row, fsl] = jnp.maximum(
                                acc_v[row, fsl], g_v[slot, e, fsl])
                    return 0

                lax.cond(same, fast, slow, 0)
                return 0

            lax.fori_loop(0, BLK // L, acc_body, 0)
            return 0

        lax.fori_loop(0, nblk, block_body, 0)

        def fin_body(r, _):
            for k in range(DL):
                sl = pl.ds(k * L, L)
                v = acc_v[r, sl]
                acc_v[r, sl] = jnp.where(v == -jnp.inf, 0.0, v)
            return 0

        lax.fori_loop(0, SEGW, fin_body, 0)
        pltpu.sync_copy(acc_v.at[pl.ds(0, SEGW)],
                        out_h.at[pl.ds(pl.multiple_of(s0, 8), SEGW)])

    return _k, SEGW, OUT_PAD, NPAD


def kernel(feat, src_ids, tgt_ids, ntypes, out_size):
    N, D = feat.shape
    E = src_ids.shape[0]
    K = ntypes.shape[1]
    try:
        OUT = int(out_size)
    except (jax.errors.ConcretizationTypeError, TypeError):
        OUT = 2500

    k, SEGW, OUT_PAD, NPAD = _build_sc_kernel(N, D, E, K, OUT)

    feat_ext = jnp.concatenate(
        [feat, jnp.full((NPAD - N, D), -jnp.inf, feat.dtype)], axis=0)
    srcp = jnp.concatenate(
        [src_ids, jnp.full((BLK,), OUT_PAD + BLK, jnp.int32)])
    tgtp = jnp.concatenate([tgt_ids, jnp.zeros((BLK,), jnp.int32)])
    ntp = jnp.concatenate(
        [ntypes.T, jnp.full((K, BLK), -1, jnp.int32)], axis=1)
    meta = jnp.concatenate([srcp[None, :], tgtp[None, :], ntp], axis=0)
    fences = jnp.arange(0, OUT_PAD + SEGW, SEGW, dtype=jnp.int32)[:NW + 1]
    bounds = jnp.searchsorted(src_ids, fences).astype(jnp.int32)
    bounds = jnp.concatenate(
        [bounds, jnp.full((48 - NW - 1,), E, jnp.int32)])

    out = k(feat_ext, meta, bounds)
    return out[:OUT]


# speedup vs baseline: 1.1905x; 1.0015x over previous
import functools

import jax
import jax.numpy as jnp
from jax import lax
from jax.experimental import pallas as pl
from jax.experimental.pallas import tpu as pltpu
from jax.experimental.pallas import tpu_sc as plsc

NC = 2
NS = 16
NW = NC * NS
L = 16
BLK = 128
GSTR = 128


def _build_sc_kernel(N, D, E, K, OUT):
    SEGW = -(-OUT // NW)
    SEGW = ((SEGW + 7) // 8) * 8
    OUT_PAD = NW * SEGW
    EP = E + BLK
    DL = D // L
    NM = 2 + K
    NSENT = N
    NPAD = -(-(N + 8) // 8) * 8
    CHB = NPAD // NS // 8 * 8
    NEX = (NPAD - CHB * NS) // 8
    mesh = plsc.VectorSubcoreMesh(core_axis_name="c", subcore_axis_name="s")

    @functools.partial(
        pl.kernel,
        out_type=jax.ShapeDtypeStruct((OUT_PAD, D), jnp.float32),
        mesh=mesh,
        scratch_types=[
            pltpu.VMEM((48,), jnp.int32),
            pltpu.VMEM((3, NM, BLK), jnp.int32),
            pltpu.VMEM((2, BLK), jnp.int32),
            pltpu.VMEM((2, BLK, D), jnp.float32),
            pltpu.VMEM((SEGW + 1, D), jnp.float32),
            pltpu.VMEM_SHARED((NPAD, D), jnp.float32),
            pltpu.SemaphoreType.DMA,
            pltpu.SemaphoreType.DMA,
        ],
    )
    def _k(feat_h, meta_h, bnd_h, out_h,
           bnd_v, m_v, idx_v, g_v, acc_v, f_sh, sem, sem2):
        cid = lax.axis_index("c")
        sid = lax.axis_index("s")
        wid = sid * NC + cid
        s0 = wid * SEGW

        off = pl.multiple_of(
            CHB * sid + 8 * jnp.minimum(sid, NEX), 8)

        @pl.when(sid < NEX)
        def _():
            pltpu.sync_copy(feat_h.at[pl.ds(off, CHB + 8)],
                            f_sh.at[pl.ds(off, CHB + 8)])

        @pl.when(sid >= NEX)
        def _():
            pltpu.sync_copy(feat_h.at[pl.ds(off, CHB)],
                            f_sh.at[pl.ds(off, CHB)])

        plsc.subcore_barrier()
        pltpu.sync_copy(bnd_h, bnd_v)
        bv = bnd_v[pl.ds(wid, L)]
        e_lo = bv[0]
        e_hi = bv[1]
        e_lo = e_lo - lax.rem(e_lo, BLK)
        nblk = lax.div(e_hi - e_lo + (BLK - 1), BLK)

        neg_inf = jnp.full((L,), -jnp.inf, jnp.float32)

        def init_body(r, _):
            for k in range(DL):
                acc_v[r, pl.ds(k * L, L)] = neg_inf
            return 0

        lax.fori_loop(0, SEGW + 1, init_body, 0)

        def meta_copy(b):
            mslot = lax.rem(b, 3)
            base = pl.multiple_of(e_lo + b * BLK, BLK)
            return pltpu.make_async_copy(
                meta_h.at[:, pl.ds(base, BLK)], m_v.at[mslot], sem2)

        def gather_fire(b, nblk):
            mslot = lax.rem(b, 3)
            slot = lax.rem(b, 2)
            meta_copy(b).wait()

            def mask_body(g, _):
                sl = pl.ds(g * L, L)
                acc = m_v[mslot, 2, sl]
                for j in range(3, NM):
                    acc = acc | m_v[mslot, j, sl]
                idx_v[slot, sl] = jnp.where(acc >= 0, m_v[mslot, 1, sl],
                                            NSENT)
                return 0

            lax.fori_loop(0, BLK // L, mask_body, 0)
            for q in range(BLK // GSTR):
                pltpu.make_async_copy(
                    f_sh.at[idx_v.at[slot].at[pl.ds(q * GSTR, GSTR)]],
                    g_v.at[slot].at[pl.ds(q * GSTR, GSTR)],
                    sem).start()

            @pl.when(b + 1 < nblk)
            def _():
                meta_copy(b + 1).start()

        @pl.when(nblk > 0)
        def _():
            meta_copy(0).start()
            gather_fire(0, nblk)

        def block_body(b, _):
            mslot = lax.rem(b, 3)
            slot = lax.rem(b, 2)

            @pl.when(b + 1 < nblk)
            def _():
                gather_fire(b + 1, nblk)

            for q in range(BLK // GSTR):
                pltpu.make_async_copy(
                    f_sh.at[idx_v.at[slot].at[pl.ds(q * GSTR, GSTR)]],
                    g_v.at[slot].at[pl.ds(q * GSTR, GSTR)],
                    sem).wait()

            def acc_body(g, _):
                sl = pl.ds(g * L, L)
                r0 = m_v[mslot, 0, sl] - s0
                rowv = jnp.where(r0 < 0, SEGW, jnp.minimum(r0, SEGW))
                row0 = rowv[0]
                same = (row0 == rowv[L - 1]) & (row0 < SEGW)

                def fast(_):
                    e0 = g * L
                    for k in range(DL):
                        fsl = pl.ds(k * L, L)
                        m = g_v[slot, e0, fsl]
                        for lane in range(1, L):
                            m = jnp.maximum(m, g_v[slot, e0 + lane, fsl])
                        acc_v[row0, fsl] = jnp.maximum(acc_v[row0, fsl], m)
                    return 0

                def slow(_):
                    for lane in range(L):
                        row = rowv[lane]
                        e = g * L + lane
                        for k in range(DL):
                            fsl = pl.ds(k * L, L)
                            acc_v[row, fsl] = jnp.maximum(
                                acc_v[row, fsl], g_v[slot, e, fsl])
                    return 0

                lax.cond(same, fast, slow, 0)
                return 0

            lax.fori_loop(0, BLK // L, acc_body, 0)
            return 0

        lax.fori_loop(0, nblk, block_body, 0)

        def fin_body(r, _):
            for k in range(DL):
                sl = pl.ds(k * L, L)
                v = acc_v[r, sl]
                acc_v[r, sl] = jnp.where(v == -jnp.inf, 0.0, v)
            return 0

        lax.fori_loop(0, SEGW, fin_body, 0)
        pltpu.sync_copy(acc_v.at[pl.ds(0, SEGW)],
                        out_h.at[pl.ds(pl.multiple_of(s0, 8), SEGW)])

    return _k, SEGW, OUT_PAD, NPAD


def kernel(feat, src_ids, tgt_ids, ntypes, out_size):
    N, D = feat.shape
    E = src_ids.shape[0]
    K = ntypes.shape[1]
    try:
        OUT = int(out_size)
    except (jax.errors.ConcretizationTypeError, TypeError):
        OUT = 2500

    k, SEGW, OUT_PAD, NPAD = _build_sc_kernel(N, D, E, K, OUT)

    feat_ext = jnp.concatenate(
        [feat, jnp.full((NPAD - N, D), -jnp.inf, feat.dtype)], axis=0)
    srcp = jnp.concatenate(
        [src_ids, jnp.full((BLK,), OUT_PAD + BLK, jnp.int32)])
    tgtp = jnp.concatenate([tgt_ids, jnp.zeros((BLK,), jnp.int32)])
    ntp = jnp.concatenate(
        [ntypes.T, jnp.full((K, BLK), -1, jnp.int32)], axis=1)
    meta = jnp.concatenate([srcp[None, :], tgtp[None, :], ntp], axis=0)
    fences = jnp.arange(0, OUT_PAD + SEGW, SEGW, dtype=jnp.int32)[:NW + 1]
    bounds = jnp.searchsorted(src_ids, fences).astype(jnp.int32)
    bounds = jnp.concatenate(
        [bounds, jnp.full((48 - NW - 1,), E, jnp.int32)])

    out = k(feat_ext, meta, bounds)
    return out[:OUT]
